# baseline (speedup 1.0000x reference)
import functools

import jax
import jax.numpy as jnp
from jax import lax
from jax.experimental import pallas as pl
from jax.experimental.pallas import tpu as pltpu
from jax.experimental.pallas import tpu_sc as plsc

N = 10000
M = 2500
NPAD = 10240
MPAD = 2560
K = 64
DF = 128
DIN = 144
HID = 256
OUT = 256
R2 = 0.01
CH = 128
NCH = NPAD // CH
S1 = 10
SLOTS = NCH * S1
TB = 256
MB = 64
EDGES = MPAD * K
BIGD = 1e9
SENT = N
FLAGCOL = DIN - 1


def _select_body(sp_ref, pos_ref, out_ref):
    P = pos_ref[...]
    S = sp_ref[...]
    pn = jnp.sum(P * P, axis=1, keepdims=True)
    sn = jnp.sum(S * S, axis=0, keepdims=True)
    d2 = pn + sn - 2.0 * jax.lax.dot_general(
        P, S, (((1,), (0,)), ((), ())), preferred_element_type=jnp.float32)
    d2v = jnp.maximum(d2, 0.0)

    ri = jax.lax.broadcasted_iota(jnp.int32, (CH, CH), 0)
    ci = jax.lax.broadcasted_iota(jnp.int32, (CH, CH), 1)
    tri = (ri >= ci).astype(jnp.float32)

    db = jax.lax.bitcast_convert_type(d2v, jnp.int32)
    colbits = jax.lax.broadcasted_iota(jnp.int32, (NPAD, TB), 0) % CH
    packed_all = jnp.bitwise_or(jnp.bitwise_and(db, -CH), colbits)
    EMPTY = jnp.int32(1 << 30)

    pk_rows = []
    for c in range(NCH):
        dc = d2v[c * CH:(c + 1) * CH, :]
        pc = packed_all[c * CH:(c + 1) * CH, :]
        mc = dc <= R2
        mcf = mc.astype(jnp.float32)
        incl = jax.lax.dot_general(
            tri, mcf, (((1,), (0,)), ((), ())),
            preferred_element_type=jnp.float32)
        ccnt = incl[CH - 1:CH, :]
        for s in range(S1):
            hit = jnp.logical_and(incl == float(s + 1), mc)
            val = jnp.sum(jnp.where(hit, pc, 0), axis=0, keepdims=True)
            occ_s = ccnt > float(s)
            pk_rows.append(jnp.where(occ_s, val, EMPTY))

    pk = jnp.concatenate(pk_rows, axis=0)
    occ = pk < EMPTY
    total = jnp.sum(occ.astype(jnp.int32), axis=0, keepdims=True)
    target = jnp.minimum(total, K)

    def bis_body(_, carry):
        lo, hi = carry
        mid = (lo + hi) // 2
        cnt_m = jnp.sum((pk <= mid).astype(jnp.int32), axis=0, keepdims=True)
        pred = cnt_m >= target
        return (jnp.where(pred, lo, mid), jnp.where(pred, mid, hi))

    lo0 = jnp.full((1, TB), -1, jnp.int32)
    hi0 = jnp.full((1, TB), 1 << 30, jnp.int32)
    _, thr = lax.fori_loop(0, 31, bis_body, (lo0, hi0))

    sel = pk <= thr
    self32 = sel.astype(jnp.float32)
    cnt = jnp.sum(sel.astype(jnp.int32), axis=0, keepdims=True)

    ri6 = jax.lax.broadcasted_iota(jnp.int32, (SLOTS, SLOTS), 0)
    ci6 = jax.lax.broadcasted_iota(jnp.int32, (SLOTS, SLOTS), 1)
    tri6 = (ri6 >= ci6).astype(jnp.float32)
    cum = jax.lax.dot_general(
        tri6, self32, (((1,), (0,)), ((), ())),
        preferred_element_type=jnp.float32)

    chunk_base = (jax.lax.broadcasted_iota(jnp.int32, (SLOTS, TB), 0)
                  // S1 * CH)
    gidx = jnp.bitwise_and(pk, CH - 1) + chunk_base

    lane = jax.lax.broadcasted_iota(jnp.int32, (1, TB), 1)
    rows = []
    for s2 in range(K):
        hit2 = jnp.logical_and(cum == float(s2 + 1), sel)
        val = jnp.sum(jnp.where(hit2, gidx, 0), axis=0, keepdims=True)
        sent = SENT + (lane + s2 * 37) % 224
        rows.append(jnp.where(cnt > s2, val, sent))
    out_ref[...] = jnp.concatenate(rows, axis=0)


def _mlp_body(g_ref, sp_ref, w1_ref, w1p_ref, b1_ref, w2_ref, b2_ref,
              out_ref):
    G = g_ref[...]
    c = b1_ref[...] - jax.lax.dot_general(
        sp_ref[...], w1p_ref[...], (((1,), (0,)), ((), ())),
        preferred_element_type=jnp.float32)
    crep = jnp.reshape(jnp.broadcast_to(c[:, None, :], (MB, K, HID)),
                       (MB * K, HID))
    h1 = jax.lax.dot_general(
        G.astype(jnp.bfloat16), w1_ref[...], (((1,), (0,)), ((), ())),
        preferred_element_type=jnp.float32)
    h1 = jnp.maximum(h1 + crep, 0.0).astype(jnp.bfloat16)
    h2 = jax.lax.dot_general(
        h1, w2_ref[...], (((1,), (0,)), ((), ())),
        preferred_element_type=jnp.float32) + b2_ref[...]
    valid = G[:, FLAGCOL:FLAGCOL + 1] > 0.5
    h2m = jnp.where(valid, h2, -1e30)
    mx = jnp.max(jnp.reshape(h2m, (MB, K, OUT)), axis=1)
    out_ref[...] = jnp.where(mx > -1e29, mx, 0.0)


def _make_sc_gather():
    info = plsc.get_sparse_core_info()
    nw = info.num_cores * info.num_subcores
    epw = EDGES // nw
    chk = 128
    grp = 256
    spg = grp // chk
    ngrp = epw // grp
    mesh = plsc.VectorSubcoreMesh(core_axis_name="c", subcore_axis_name="s")

    @functools.partial(
        pl.kernel, mesh=mesh,
        compiler_params=pltpu.CompilerParams(use_tc_tiling_on_sc=False),
        out_type=jax.ShapeDtypeStruct((EDGES, DIN), jnp.float32),
        scratch_types=[
            pltpu.VMEM((epw,), jnp.int32),
            pltpu.VMEM((grp, DIN), jnp.float32),
            pltpu.VMEM((grp, DIN), jnp.float32),
            pltpu.SemaphoreType.DMA,
            pltpu.SemaphoreType.DMA,
            pltpu.SemaphoreType.DMA,
            pltpu.SemaphoreType.DMA,
        ],
    )
    def sc_gather(table_hbm, idx_hbm, out_hbm, idx_v, rows0, rows1,
                  sg0, sg1, ss0, ss1):
        wid = lax.axis_index("s") * info.num_cores + lax.axis_index("c")
        base = wid * epw
        pltpu.sync_copy(idx_hbm.at[pl.ds(base, epw)], idx_v)
        bufs = (rows0, rows1)
        sgs = (sg0, sg1)
        sss = (ss0, ss1)

        def start_gather(g, b):
            hs = []
            for j in range(spg):
                hs.append(pltpu.async_copy(
                    table_hbm.at[idx_v.at[pl.ds(g * grp + j * chk, chk)]],
                    bufs[b].at[pl.ds(j * chk, chk)], sgs[b]))
            return hs

        gh = {0: start_gather(0, 0)}
        sh = {}
        for g in range(ngrp):
            b = g & 1
            if g + 1 < ngrp:
                if g >= 1:
                    sh.pop(g - 1).wait()
                gh[g + 1] = start_gather(g + 1, b ^ 1)
            for h in gh.pop(g):
                h.wait()
            sh[g] = pltpu.async_copy(
                bufs[b], out_hbm.at[pl.ds(base + g * grp, grp)], sss[b])
        sh.pop(ngrp - 2).wait()
        sh.pop(ngrp - 1).wait()

    return sc_gather


def kernel(feat, sampled_feat, pos, sampled_pos, batch, W1, b1, W2, b2):
    f32 = jnp.float32
    pos_pad = jnp.concatenate(
        [pos, jnp.full((NPAD - N, 3), 1e3, f32)], axis=0)
    pos8 = jnp.concatenate(
        [pos_pad, jnp.zeros((NPAD, 5), f32)], axis=1)
    sp_pad = jnp.concatenate(
        [sampled_pos, jnp.full((MPAD - M, 3), -1e3, f32)], axis=0)
    sp8 = jnp.concatenate(
        [sp_pad, jnp.zeros((MPAD, 5), f32)], axis=1)
    sp8T = sp8.T

    flag = jnp.ones((N, 1), f32)
    table = jnp.concatenate(
        [feat, pos, jnp.zeros((N, DIN - DF - 3 - 1), f32), flag], axis=1)
    table = jnp.concatenate(
        [table, jnp.zeros((NPAD - N, DIN), f32)],
        axis=0)

    W1p = jnp.concatenate(
        [W1, jnp.zeros((DIN - DF - 3, HID), f32)],
        axis=0).astype(jnp.bfloat16)
    W1pos8 = jnp.concatenate([W1[DF:DF + 3], jnp.zeros((5, HID), f32)],
                             axis=0)
    b1r = b1.reshape(1, HID)
    b2r = b2.reshape(1, OUT)

    nbr_t = pl.pallas_call(
        _select_body,
        grid=(MPAD // TB,),
        in_specs=[
            pl.BlockSpec((8, TB), lambda i: (0, i)),
            pl.BlockSpec((NPAD, 8), lambda i: (0, 0)),
        ],
        out_specs=pl.BlockSpec((K, TB), lambda i: (0, i)),
        out_shape=jax.ShapeDtypeStruct((K, MPAD), jnp.int32),
    )(sp8T, pos8)

    edge_idx = jnp.transpose(nbr_t).reshape(EDGES)

    gathered = _make_sc_gather()(table, edge_idx)

    out_pad = pl.pallas_call(
        _mlp_body,
        grid=(MPAD // MB,),
        in_specs=[
            pl.BlockSpec((MB * K, DIN), lambda i: (i, 0)),
            pl.BlockSpec((MB, 8), lambda i: (i, 0)),
            pl.BlockSpec((DIN, HID), lambda i: (0, 0)),
            pl.BlockSpec((8, HID), lambda i: (0, 0)),
            pl.BlockSpec((1, HID), lambda i: (0, 0)),
            pl.BlockSpec((HID, OUT), lambda i: (0, 0)),
            pl.BlockSpec((1, OUT), lambda i: (0, 0)),
        ],
        out_specs=pl.BlockSpec((MB, OUT), lambda i: (i, 0)),
        out_shape=jax.ShapeDtypeStruct((MPAD, OUT), f32),
    )(gathered, sp8, W1p, W1pos8, b1r, W2.astype(jnp.bfloat16), b2r)

    return (out_pad[:M], sampled_pos, batch)


# speedup vs baseline: 1.0758x; 1.0758x over previous
import functools

import jax
import jax.numpy as jnp
from jax import lax
from jax.experimental import pallas as pl
from jax.experimental.pallas import tpu as pltpu
from jax.experimental.pallas import tpu_sc as plsc

N = 10000
M = 2500
NPAD = 10240
MPAD = 2560
K = 64
DF = 128
DIN = 144
HID = 256
OUT = 256
R2 = 0.01
CH = 128
NCH = NPAD // CH
S1 = 10
SLOTS = NCH * S1
TB = 128
MB = 64
EDGES = MPAD * K
BIGD = 1e9
SENT = N
FLAGCOL = DIN - 1


def _select_body(sp_ref, pos_ref, out_ref):
    P = pos_ref[...]
    S = sp_ref[...]
    pn = jnp.sum(P * P, axis=1, keepdims=True)
    sn = jnp.sum(S * S, axis=0, keepdims=True)
    d2 = pn + sn - 2.0 * jax.lax.dot_general(
        P, S, (((1,), (0,)), ((), ())), preferred_element_type=jnp.float32)
    d2v = jnp.maximum(d2, 0.0)

    ri = jax.lax.broadcasted_iota(jnp.int32, (CH, CH), 0)
    ci = jax.lax.broadcasted_iota(jnp.int32, (CH, CH), 1)
    tri = (ri >= ci).astype(jnp.float32)

    db = jax.lax.bitcast_convert_type(d2v, jnp.int32)
    colbits = jax.lax.broadcasted_iota(jnp.int32, (NPAD, TB), 0) % CH
    packed_all = jnp.bitwise_or(jnp.bitwise_and(db, -CH), colbits)
    EMPTY = jnp.int32(1 << 30)

    pk_rows = []
    for c in range(NCH):
        dc = d2v[c * CH:(c + 1) * CH, :]
        pc = packed_all[c * CH:(c + 1) * CH, :]
        mc = dc <= R2
        mcf = mc.astype(jnp.float32)
        incl = jax.lax.dot_general(
            tri, mcf, (((1,), (0,)), ((), ())),
            preferred_element_type=jnp.float32)
        ccnt = incl[CH - 1:CH, :]
        for s in range(S1):
            hit = jnp.logical_and(incl == float(s + 1), mc)
            val = jnp.sum(jnp.where(hit, pc, 0), axis=0, keepdims=True)
            occ_s = ccnt > float(s)
            pk_rows.append(jnp.where(occ_s, val, EMPTY))

    pk = jnp.concatenate(pk_rows, axis=0)
    occ = pk < EMPTY
    total = jnp.sum(occ.astype(jnp.int32), axis=0, keepdims=True)
    target = jnp.minimum(total, K)

    def bis_body(_, carry):
        lo, hi = carry
        mid = (lo + hi) // 2
        cnt_m = jnp.sum((pk <= mid).astype(jnp.int32), axis=0, keepdims=True)
        pred = cnt_m >= target
        return (jnp.where(pred, lo, mid), jnp.where(pred, mid, hi))

    lo0 = jnp.full((1, TB), -1, jnp.int32)
    hi0 = jnp.full((1, TB), 1 << 30, jnp.int32)
    _, thr = lax.fori_loop(0, 31, bis_body, (lo0, hi0))

    sel = pk <= thr
    self32 = sel.astype(jnp.float32)
    cnt = jnp.sum(sel.astype(jnp.int32), axis=0, keepdims=True)

    ri6 = jax.lax.broadcasted_iota(jnp.int32, (SLOTS, SLOTS), 0)
    ci6 = jax.lax.broadcasted_iota(jnp.int32, (SLOTS, SLOTS), 1)
    tri6 = (ri6 >= ci6).astype(jnp.float32)
    cum = jax.lax.dot_general(
        tri6, self32, (((1,), (0,)), ((), ())),
        preferred_element_type=jnp.float32)

    chunk_base = (jax.lax.broadcasted_iota(jnp.int32, (SLOTS, TB), 0)
                  // S1 * CH)
    gidx = jnp.bitwise_and(pk, CH - 1) + chunk_base

    lane = jax.lax.broadcasted_iota(jnp.int32, (1, TB), 1)
    rows = []
    for s2 in range(K):
        hit2 = jnp.logical_and(cum == float(s2 + 1), sel)
        val = jnp.sum(jnp.where(hit2, gidx, 0), axis=0, keepdims=True)
        sent = SENT + (lane + s2 * 37) % 224
        rows.append(jnp.where(cnt > s2, val, sent))
    out_ref[...] = jnp.concatenate(rows, axis=0)


def _mlp_body(g_ref, sp_ref, w1_ref, w1p_ref, b1_ref, w2_ref, b2_ref,
              out_ref):
    G = g_ref[...]
    c = b1_ref[...] - jax.lax.dot_general(
        sp_ref[...], w1p_ref[...], (((1,), (0,)), ((), ())),
        preferred_element_type=jnp.float32)
    crep = jnp.reshape(jnp.broadcast_to(c[:, None, :], (MB, K, HID)),
                       (MB * K, HID))
    h1 = jax.lax.dot_general(
        G.astype(jnp.bfloat16), w1_ref[...], (((1,), (0,)), ((), ())),
        preferred_element_type=jnp.float32)
    h1 = jnp.maximum(h1 + crep, 0.0).astype(jnp.bfloat16)
    h2 = jax.lax.dot_general(
        h1, w2_ref[...], (((1,), (0,)), ((), ())),
        preferred_element_type=jnp.float32) + b2_ref[...]
    valid = G[:, FLAGCOL:FLAGCOL + 1] > 0.5
    h2m = jnp.where(valid, h2, -1e30)
    mx = jnp.max(jnp.reshape(h2m, (MB, K, OUT)), axis=1)
    out_ref[...] = jnp.where(mx > -1e29, mx, 0.0)


def _make_sc_gather():
    info = plsc.get_sparse_core_info()
    nw = info.num_cores * info.num_subcores
    epw = EDGES // nw
    chk = 128
    grp = 256
    spg = grp // chk
    ngrp = epw // grp
    mesh = plsc.VectorSubcoreMesh(core_axis_name="c", subcore_axis_name="s")

    @functools.partial(
        pl.kernel, mesh=mesh,
        compiler_params=pltpu.CompilerParams(use_tc_tiling_on_sc=False),
        out_type=jax.ShapeDtypeStruct((EDGES, DIN), jnp.float32),
        scratch_types=[
            pltpu.VMEM((epw,), jnp.int32),
            pltpu.VMEM((grp, DIN), jnp.float32),
            pltpu.VMEM((grp, DIN), jnp.float32),
            pltpu.SemaphoreType.DMA,
            pltpu.SemaphoreType.DMA,
            pltpu.SemaphoreType.DMA,
            pltpu.SemaphoreType.DMA,
        ],
    )
    def sc_gather(table_hbm, idx_hbm, out_hbm, idx_v, rows0, rows1,
                  sg0, sg1, ss0, ss1):
        wid = lax.axis_index("s") * info.num_cores + lax.axis_index("c")
        base = wid * epw
        pltpu.sync_copy(idx_hbm.at[pl.ds(base, epw)], idx_v)
        bufs = (rows0, rows1)
        sgs = (sg0, sg1)
        sss = (ss0, ss1)

        def start_gather(g, b):
            hs = []
            for j in range(spg):
                hs.append(pltpu.async_copy(
                    table_hbm.at[idx_v.at[pl.ds(g * grp + j * chk, chk)]],
                    bufs[b].at[pl.ds(j * chk, chk)], sgs[b]))
            return hs

        gh = {0: start_gather(0, 0)}
        sh = {}
        for g in range(ngrp):
            b = g & 1
            if g + 1 < ngrp:
                if g >= 1:
                    sh.pop(g - 1).wait()
                gh[g + 1] = start_gather(g + 1, b ^ 1)
            for h in gh.pop(g):
                h.wait()
            sh[g] = pltpu.async_copy(
                bufs[b], out_hbm.at[pl.ds(base + g * grp, grp)], sss[b])
        sh.pop(ngrp - 2).wait()
        sh.pop(ngrp - 1).wait()

    return sc_gather


def kernel(feat, sampled_feat, pos, sampled_pos, batch, W1, b1, W2, b2):
    f32 = jnp.float32
    pos_pad = jnp.concatenate(
        [pos, jnp.full((NPAD - N, 3), 1e3, f32)], axis=0)
    pos8 = jnp.concatenate(
        [pos_pad, jnp.zeros((NPAD, 5), f32)], axis=1)
    sp_pad = jnp.concatenate(
        [sampled_pos, jnp.full((MPAD - M, 3), -1e3, f32)], axis=0)
    sp8 = jnp.concatenate(
        [sp_pad, jnp.zeros((MPAD, 5), f32)], axis=1)
    sp8T = sp8.T

    flag = jnp.ones((N, 1), f32)
    table = jnp.concatenate(
        [feat, pos, jnp.zeros((N, DIN - DF - 3 - 1), f32), flag], axis=1)
    table = jnp.concatenate(
        [table, jnp.zeros((NPAD - N, DIN), f32)],
        axis=0)

    W1p = jnp.concatenate(
        [W1, jnp.zeros((DIN - DF - 3, HID), f32)],
        axis=0).astype(jnp.bfloat16)
    W1pos8 = jnp.concatenate([W1[DF:DF + 3], jnp.zeros((5, HID), f32)],
                             axis=0)
    b1r = b1.reshape(1, HID)
    b2r = b2.reshape(1, OUT)

    nbr_t = pl.pallas_call(
        _select_body,
        grid=(MPAD // TB,),
        in_specs=[
            pl.BlockSpec((8, TB), lambda i: (0, i)),
            pl.BlockSpec((NPAD, 8), lambda i: (0, 0)),
        ],
        out_specs=pl.BlockSpec((K, TB), lambda i: (0, i)),
        out_shape=jax.ShapeDtypeStruct((K, MPAD), jnp.int32),
    )(sp8T, pos8)

    edge_idx = jnp.transpose(nbr_t).reshape(EDGES)

    gathered = _make_sc_gather()(table, edge_idx)

    out_pad = pl.pallas_call(
        _mlp_body,
        grid=(MPAD // MB,),
        in_specs=[
            pl.BlockSpec((MB * K, DIN), lambda i: (i, 0)),
            pl.BlockSpec((MB, 8), lambda i: (i, 0)),
            pl.BlockSpec((DIN, HID), lambda i: (0, 0)),
            pl.BlockSpec((8, HID), lambda i: (0, 0)),
            pl.BlockSpec((1, HID), lambda i: (0, 0)),
            pl.BlockSpec((HID, OUT), lambda i: (0, 0)),
            pl.BlockSpec((1, OUT), lambda i: (0, 0)),
        ],
        out_specs=pl.BlockSpec((MB, OUT), lambda i: (i, 0)),
        out_shape=jax.ShapeDtypeStruct((MPAD, OUT), f32),
    )(gathered, sp8, W1p, W1pos8, b1r, W2.astype(jnp.bfloat16), b2r)

    return (out_pad[:M], sampled_pos, batch)


# speedup vs baseline: 1.1614x; 1.0796x over previous
import functools

import jax
import jax.numpy as jnp
from jax import lax
from jax.experimental import pallas as pl
from jax.experimental.pallas import tpu as pltpu
from jax.experimental.pallas import tpu_sc as plsc

N = 10000
M = 2500
NPAD = 10240
MPAD = 2560
K = 64
DF = 128
DIN = 144
HID = 256
OUT = 256
R2 = 0.01
CH = 128
NCH = NPAD // CH
S1 = 10
SLOTS = NCH * S1
TB = 128
MB = 64
EDGES = MPAD * K
BIGD = 1e9
SENT = N
FLAGCOL = DIN - 1


def _select_body(sp_ref, pos_ref, out_ref):
    P = pos_ref[...]
    S = sp_ref[...]
    pn = jnp.sum(P * P, axis=1, keepdims=True)
    sn = jnp.sum(S * S, axis=0, keepdims=True)
    d2 = pn + sn - 2.0 * jax.lax.dot_general(
        P, S, (((1,), (0,)), ((), ())), preferred_element_type=jnp.float32)
    d2v = jnp.maximum(d2, 0.0)

    ri = jax.lax.broadcasted_iota(jnp.int32, (CH, CH), 0)
    ci = jax.lax.broadcasted_iota(jnp.int32, (CH, CH), 1)
    tri = (ri >= ci).astype(jnp.float32)

    db = jax.lax.bitcast_convert_type(d2v, jnp.int32)
    colbits = jax.lax.broadcasted_iota(jnp.int32, (NPAD, TB), 0) % CH
    packed_all = jnp.bitwise_or(jnp.bitwise_and(db, -CH), colbits)
    EMPTY = jnp.int32(1 << 30)

    pk_rows = []
    for c in range(NCH):
        dc = d2v[c * CH:(c + 1) * CH, :]
        pc = packed_all[c * CH:(c + 1) * CH, :]
        mc = dc <= R2
        mcf = mc.astype(jnp.float32)
        incl = jax.lax.dot_general(
            tri, mcf, (((1,), (0,)), ((), ())),
            preferred_element_type=jnp.float32)
        ccnt = incl[CH - 1:CH, :]
        for s in range(S1):
            hit = jnp.logical_and(incl == float(s + 1), mc)
            val = jnp.sum(jnp.where(hit, pc, 0), axis=0, keepdims=True)
            occ_s = ccnt > float(s)
            pk_rows.append(jnp.where(occ_s, val, EMPTY))

    pk = jnp.concatenate(pk_rows, axis=0)
    occ = pk < EMPTY
    total = jnp.sum(occ.astype(jnp.int32), axis=0, keepdims=True)
    target = jnp.minimum(total, K)

    def bis_body(_, carry):
        lo, hi = carry
        mid = (lo + hi) // 2
        cnt_m = jnp.sum((pk <= mid).astype(jnp.int32), axis=0, keepdims=True)
        pred = cnt_m >= target
        return (jnp.where(pred, lo, mid), jnp.where(pred, mid, hi))

    lo0 = jnp.full((1, TB), -1, jnp.int32)
    hi0 = jnp.full((1, TB), 1 << 30, jnp.int32)
    _, thr = lax.fori_loop(0, 31, bis_body, (lo0, hi0))

    sel = pk <= thr
    self32 = sel.astype(jnp.float32)
    cnt = jnp.sum(sel.astype(jnp.int32), axis=0, keepdims=True)

    ri6 = jax.lax.broadcasted_iota(jnp.int32, (SLOTS, SLOTS), 0)
    ci6 = jax.lax.broadcasted_iota(jnp.int32, (SLOTS, SLOTS), 1)
    tri6 = (ri6 >= ci6).astype(jnp.float32)
    cum = jax.lax.dot_general(
        tri6, self32, (((1,), (0,)), ((), ())),
        preferred_element_type=jnp.float32)

    chunk_base = (jax.lax.broadcasted_iota(jnp.int32, (SLOTS, TB), 0)
                  // S1 * CH)
    gidx = jnp.bitwise_and(pk, CH - 1) + chunk_base

    lane = jax.lax.broadcasted_iota(jnp.int32, (1, TB), 1)
    rows = []
    for s2 in range(K):
        hit2 = jnp.logical_and(cum == float(s2 + 1), sel)
        val = jnp.sum(jnp.where(hit2, gidx, 0), axis=0, keepdims=True)
        sent = SENT + (lane + s2 * 37) % 224
        rows.append(jnp.where(cnt > s2, val, sent))
    out_ref[...] = jnp.concatenate(rows, axis=0)


def _mlp_body(g_ref, sp_ref, w1_ref, w1p_ref, b1_ref, w2_ref, b2_ref,
              out_ref):
    G = g_ref[...]
    c = b1_ref[...] - jax.lax.dot_general(
        sp_ref[...], w1p_ref[...], (((1,), (0,)), ((), ())),
        preferred_element_type=jnp.float32)
    crep = jnp.reshape(jnp.broadcast_to(c[:, None, :], (MB, K, HID)),
                       (MB * K, HID))
    h1 = jax.lax.dot_general(
        G.astype(jnp.bfloat16), w1_ref[...], (((1,), (0,)), ((), ())),
        preferred_element_type=jnp.float32)
    h1 = jnp.maximum(h1 + crep, 0.0).astype(jnp.bfloat16)
    h2 = jax.lax.dot_general(
        h1, w2_ref[...], (((1,), (0,)), ((), ())),
        preferred_element_type=jnp.float32) + b2_ref[...]
    valid = G[:, FLAGCOL:FLAGCOL + 1] > 0.5
    h2m = jnp.where(valid, h2, -1e30)
    mx = jnp.max(jnp.reshape(h2m, (MB, K, OUT)), axis=1)
    out_ref[...] = jnp.where(mx > -1e29, mx, 0.0)


def _make_sc_gather(edges):
    info = plsc.get_sparse_core_info()
    nw = info.num_cores * info.num_subcores
    epw = edges // nw
    chk = 128
    grp = 256
    spg = grp // chk
    ngrp = epw // grp
    mesh = plsc.VectorSubcoreMesh(core_axis_name="c", subcore_axis_name="s")

    @functools.partial(
        pl.kernel, mesh=mesh,
        compiler_params=pltpu.CompilerParams(use_tc_tiling_on_sc=False),
        out_type=jax.ShapeDtypeStruct((edges, DIN), jnp.float32),
        scratch_types=[
            pltpu.VMEM((epw,), jnp.int32),
            pltpu.VMEM((grp, DIN), jnp.float32),
            pltpu.VMEM((grp, DIN), jnp.float32),
            pltpu.SemaphoreType.DMA,
            pltpu.SemaphoreType.DMA,
            pltpu.SemaphoreType.DMA,
            pltpu.SemaphoreType.DMA,
        ],
    )
    def sc_gather(table_hbm, idx_hbm, out_hbm, idx_v, rows0, rows1,
                  sg0, sg1, ss0, ss1):
        wid = lax.axis_index("s") * info.num_cores + lax.axis_index("c")
        base = wid * epw
        pltpu.sync_copy(idx_hbm.at[pl.ds(base, epw)], idx_v)
        bufs = (rows0, rows1)
        sgs = (sg0, sg1)
        sss = (ss0, ss1)

        def start_gather(g, b):
            hs = []
            for j in range(spg):
                hs.append(pltpu.async_copy(
                    table_hbm.at[idx_v.at[pl.ds(g * grp + j * chk, chk)]],
                    bufs[b].at[pl.ds(j * chk, chk)], sgs[b]))
            return hs

        gh = {0: start_gather(0, 0)}
        sh = {}
        for g in range(ngrp):
            b = g & 1
            if g + 1 < ngrp:
                if g >= 1:
                    sh.pop(g - 1).wait()
                gh[g + 1] = start_gather(g + 1, b ^ 1)
            for h in gh.pop(g):
                h.wait()
            sh[g] = pltpu.async_copy(
                bufs[b], out_hbm.at[pl.ds(base + g * grp, grp)], sss[b])
        sh.pop(ngrp - 2).wait()
        sh.pop(ngrp - 1).wait()

    return sc_gather


def kernel(feat, sampled_feat, pos, sampled_pos, batch, W1, b1, W2, b2):
    f32 = jnp.float32
    pos_pad = jnp.concatenate(
        [pos, jnp.full((NPAD - N, 3), 1e3, f32)], axis=0)
    pos8 = jnp.concatenate(
        [pos_pad, jnp.zeros((NPAD, 5), f32)], axis=1)
    sp_pad = jnp.concatenate(
        [sampled_pos, jnp.full((MPAD - M, 3), -1e3, f32)], axis=0)
    sp8 = jnp.concatenate(
        [sp_pad, jnp.zeros((MPAD, 5), f32)], axis=1)
    sp8T = sp8.T

    flag = jnp.ones((N, 1), f32)
    table = jnp.concatenate(
        [feat, pos, jnp.zeros((N, DIN - DF - 3 - 1), f32), flag], axis=1)
    table = jnp.concatenate(
        [table, jnp.zeros((NPAD - N, DIN), f32)],
        axis=0)

    W1p = jnp.concatenate(
        [W1, jnp.zeros((DIN - DF - 3, HID), f32)],
        axis=0).astype(jnp.bfloat16)
    W1pos8 = jnp.concatenate([W1[DF:DF + 3], jnp.zeros((5, HID), f32)],
                             axis=0)
    b1r = b1.reshape(1, HID)
    b2r = b2.reshape(1, OUT)

    MH = MPAD // 2
    EH = MH * K
    sc_gather = _make_sc_gather(EH)
    w2b = W2.astype(jnp.bfloat16)
    outs = []
    for h in range(2):
        sp8T_h = sp8T[:, h * MH:(h + 1) * MH]
        sp8_h = sp8[h * MH:(h + 1) * MH]
        nbr_t = pl.pallas_call(
            _select_body,
            grid=(MH // TB,),
            in_specs=[
                pl.BlockSpec((8, TB), lambda i: (0, i)),
                pl.BlockSpec((NPAD, 8), lambda i: (0, 0)),
            ],
            out_specs=pl.BlockSpec((K, TB), lambda i: (0, i)),
            out_shape=jax.ShapeDtypeStruct((K, MH), jnp.int32),
        )(sp8T_h, pos8)
        edge_idx = jnp.transpose(nbr_t).reshape(EH)
        gathered = sc_gather(table, edge_idx)
        out_h = pl.pallas_call(
            _mlp_body,
            grid=(MH // MB,),
            in_specs=[
                pl.BlockSpec((MB * K, DIN), lambda i: (i, 0)),
                pl.BlockSpec((MB, 8), lambda i: (i, 0)),
                pl.BlockSpec((DIN, HID), lambda i: (0, 0)),
                pl.BlockSpec((8, HID), lambda i: (0, 0)),
                pl.BlockSpec((1, HID), lambda i: (0, 0)),
                pl.BlockSpec((HID, OUT), lambda i: (0, 0)),
                pl.BlockSpec((1, OUT), lambda i: (0, 0)),
            ],
            out_specs=pl.BlockSpec((MB, OUT), lambda i: (i, 0)),
            out_shape=jax.ShapeDtypeStruct((MH, OUT), f32),
        )(gathered, sp8_h, W1p, W1pos8, b1r, w2b, b2r)
        outs.append(out_h)

    out_pad = jnp.concatenate(outs, axis=0)
    return (out_pad[:M], sampled_pos, batch)
